# Initial kernel scaffold; baseline (speedup 1.0000x reference)
import jax
import jax.numpy as jnp
from jax.experimental import pallas as pl


def kernel(X, rand_vals, emb, W, b):
    raise NotImplementedError("write your pallas kernel here")


# speedup vs baseline: 10.4784x; 10.4784x over previous
import jax
import jax.numpy as jnp
from jax.experimental import pallas as pl
from jax.experimental.pallas import tpu as pltpu

_VOCAB = 8192
_D = 768
_ALPHA = 0.1
_TILE = 1024
_MASK_TOK = 2


def _body(emb_ref, W_ref, b_ref, X_ref, rand_ref, out_ref, bestv_ref, besti_ref):
    j = pl.program_id(0)
    nj = pl.num_programs(0)

    v = emb_ref[_MASK_TOK : _MASK_TOK + 1, :]
    s = (
        jax.lax.dot_general(
            v, W_ref[...], (((1,), (0,)), ((), ())),
            preferred_element_type=jnp.float32,
        )
        + b_ref[...]
    )

    m = jnp.max(s)
    idx = jax.lax.broadcasted_iota(jnp.int32, s.shape, 1)
    a = jnp.min(jnp.where(s == m, idx, _TILE))

    @pl.when(j == 0)
    def _():
        bestv_ref[0] = m
        besti_ref[0] = a

    @pl.when((j > 0) & (m > bestv_ref[0]))
    def _():
        bestv_ref[0] = m
        besti_ref[0] = j * _TILE + a

    @pl.when(j == nj - 1)
    def _():
        p = besti_ref[0].astype(jnp.float32)
        X = X_ref[...]
        cond = (X == jnp.float32(_MASK_TOK)) & (rand_ref[...] < jnp.float32(_ALPHA))
        out_ref[...] = jnp.where(cond, p, X)


def kernel(X, rand_vals, emb, W, b):
    Bsz, L = X.shape
    b2 = b.reshape(1, _VOCAB)
    out = pl.pallas_call(
        _body,
        grid=(_VOCAB // _TILE,),
        in_specs=[
            pl.BlockSpec((8, _D), lambda j: (0, 0)),
            pl.BlockSpec((_D, _TILE), lambda j: (0, j)),
            pl.BlockSpec((1, _TILE), lambda j: (0, j)),
            pl.BlockSpec((Bsz, L), lambda j: (0, 0)),
            pl.BlockSpec((Bsz, L), lambda j: (0, 0)),
        ],
        out_specs=pl.BlockSpec((Bsz, L), lambda j: (0, 0)),
        out_shape=jax.ShapeDtypeStruct((Bsz, L), X.dtype),
        scratch_shapes=[
            pltpu.SMEM((1,), jnp.float32),
            pltpu.SMEM((1,), jnp.int32),
        ],
    )(emb, W, b2, X, rand_vals)
    return out
